# baseline (speedup 1.0000x reference)
import functools

import jax
import jax.numpy as jnp
from jax import lax
from jax.experimental import pallas as pl
from jax.experimental.pallas import tpu as pltpu
from jax.experimental.pallas import tpu_sc as plsc

_N = 100000
_D = 128
_T_ROWS = 457
_NC = 2
_NS = 16
_NW = _NC * _NS
_OCT = _N // 8 // _NW
_OCT_EXTRA = _N // 8 - _OCT * _NW
_CHUNK = 32
_NPAIRS = 49
_XROWS = 3136
_WDIMS = (5, 12, 12, 10, 6, 6, 2, 2)
_WOFF = []
_acc = 0
for _d in _WDIMS:
  _WOFF.append(_acc * _D)
  _acc += _d
_WSIZE = _acc * _D

_BCAST_DN = lax.GatherDimensionNumbers(
    offset_dims=(), collapsed_slice_dims=(0,), start_index_map=(0,))


def _bcast(vec, r):
  return lax.gather(vec, jnp.full((16, 1), r, jnp.int32), _BCAST_DN, (1,),
                    mode=lax.GatherScatterMode.PROMISE_IN_BOUNDS)


@functools.partial(
    pl.kernel,
    out_type=jax.ShapeDtypeStruct((_N * _D,), jnp.float32),
    mesh=plsc.VectorSubcoreMesh(core_axis_name="c", subcore_axis_name="s"),
    compiler_params=pltpu.CompilerParams(needs_layout_passes=False),
    scratch_types=[
        pltpu.VMEM((_T_ROWS * _D,), jnp.float32),
        pltpu.VMEM((_WSIZE,), jnp.float32),
        pltpu.VMEM((_XROWS * 9,), jnp.int32),
        pltpu.VMEM((_CHUNK * _D,), jnp.float32),
        pltpu.VMEM((_CHUNK * _D,), jnp.float32),
        pltpu.SemaphoreType.DMA,
        pltpu.SemaphoreType.DMA,
    ],
)
def _sc_lookup(x_hbm, w0_hbm, w1_hbm, w2_hbm, w3_hbm, w4_hbm, w5_hbm,
               w6_hbm, w7_hbm, w8_hbm, out_hbm, t_v, w_v, x_v, o_a, o_b,
               sem_a, sem_b):
  wid = lax.axis_index("s") * _NC + lax.axis_index("c")
  w = wid.astype(jnp.int32)
  start8 = w * _OCT + jnp.minimum(w, _OCT_EXTRA)
  n8 = _OCT + (w < _OCT_EXTRA).astype(jnp.int32)
  wstart = start8 * 8
  wlast = wstart + n8 * 8 - _CHUNK
  xbase = jnp.minimum(wstart, _N - _XROWS)

  pltpu.sync_copy(x_hbm.at[pl.ds(xbase * 9, _XROWS * 9)], x_v)
  pltpu.sync_copy(w0_hbm, t_v.at[pl.ds(0, 119 * _D)])
  for i, wh in enumerate((w1_hbm, w2_hbm, w3_hbm, w4_hbm,
                          w5_hbm, w6_hbm, w7_hbm, w8_hbm)):
    pltpu.sync_copy(wh, w_v.at[pl.ds(_WOFF[i], _WDIMS[i] * _D)])

  lane = lax.iota(jnp.int32, 16)
  lane9 = lane * 9
  coffs = [lane + 16 * c8 for c8 in range(_D // 16)]

  def g1_row(r, carry):
    p = r // 10
    q = r % 10
    s1 = _WOFF[0] + p * _D
    s4 = _WOFF[3] + q * _D
    dst = (119 + r) * _D
    for c in range(0, _D, 16):
      t_v[pl.ds(dst + c, 16)] = (
          w_v[pl.ds(s1 + c, 16)] + w_v[pl.ds(s4 + c, 16)])
    return carry

  def g2_row(r, carry):
    p = r // 12
    q = r % 12
    s2 = _WOFF[1] + p * _D
    s3 = _WOFF[2] + q * _D
    dst = (169 + r) * _D
    for c in range(0, _D, 16):
      t_v[pl.ds(dst + c, 16)] = (
          w_v[pl.ds(s2 + c, 16)] + w_v[pl.ds(s3 + c, 16)])
    return carry

  def g3_row(r, carry):
    a = r // 24
    e = r % 24
    b = e // 4
    e2 = e % 4
    c_ = e2 // 2
    d = e2 % 2
    s5 = _WOFF[4] + a * _D
    s6 = _WOFF[5] + b * _D
    s7 = _WOFF[6] + c_ * _D
    s8 = _WOFF[7] + d * _D
    dst = (313 + r) * _D
    for c in range(0, _D, 16):
      t_v[pl.ds(dst + c, 16)] = (
          (w_v[pl.ds(s5 + c, 16)] + w_v[pl.ds(s6 + c, 16)]) +
          (w_v[pl.ds(s7 + c, 16)] + w_v[pl.ds(s8 + c, 16)]))
    return carry

  lax.fori_loop(0, 50, g1_row, 0)
  lax.fori_loop(0, 144, g2_row, 0)
  lax.fori_loop(0, 144, g3_row, 0)

  def do_subchunk(base, r0, o_ref):
    xoff = (base + r0 - xbase) * 9
    xv = [plsc.load_gather(x_v, [xoff + lane9 + j]) for j in range(9)]
    i0 = xv[0]
    i1 = 119 + xv[1] * 10 + xv[4]
    i2 = 169 + xv[2] * 12 + xv[3]
    i3 = 313 + xv[5] * 24 + xv[6] * 4 + xv[7] * 2 + xv[8]
    a0 = i0 * _D
    a1 = i1 * _D
    a2 = i2 * _D
    a3 = i3 * _D
    def row_body(r, carry):
      ridx = jnp.full((16, 1), r, jnp.int32)
      b0 = lax.gather(a0, ridx, _BCAST_DN, (1,),
                      mode=lax.GatherScatterMode.PROMISE_IN_BOUNDS)
      b1 = lax.gather(a1, ridx, _BCAST_DN, (1,),
                      mode=lax.GatherScatterMode.PROMISE_IN_BOUNDS)
      b2 = lax.gather(a2, ridx, _BCAST_DN, (1,),
                      mode=lax.GatherScatterMode.PROMISE_IN_BOUNDS)
      b3 = lax.gather(a3, ridx, _BCAST_DN, (1,),
                      mode=lax.GatherScatterMode.PROMISE_IN_BOUNDS)
      quads = []
      for co in coffs:
        quads.append((plsc.load_gather(t_v, [b0 + co]),
                      plsc.load_gather(t_v, [b1 + co]),
                      plsc.load_gather(t_v, [b2 + co]),
                      plsc.load_gather(t_v, [b3 + co])))
      off = (r0 + r) * _D
      for c8 in range(_D // 16):
        q = quads[c8]
        o_ref[pl.ds(off + c8 * 16, 16)] = (q[0] + q[1]) + (q[2] + q[3])
      return carry

    lax.fori_loop(0, 16, row_body, 0)

  def do_chunk(i, o_ref):
    base = jnp.minimum(wstart + i * _CHUNK, wlast)
    for r0 in range(0, _CHUNK, 16):
      do_subchunk(base, r0, o_ref)
    return base

  def body(k2, carry):
    @pl.when(k2 > 0)
    def _():
      pltpu.make_async_copy(
          o_a, out_hbm.at[pl.ds(0, _CHUNK * _D)], sem_a).wait()
    base_a = do_chunk(k2 * 2, o_a)
    pltpu.async_copy(o_a, out_hbm.at[pl.ds(base_a * _D, _CHUNK * _D)], sem_a)

    @pl.when(k2 > 0)
    def _():
      pltpu.make_async_copy(
          o_b, out_hbm.at[pl.ds(0, _CHUNK * _D)], sem_b).wait()
    base_b = do_chunk(k2 * 2 + 1, o_b)
    pltpu.async_copy(o_b, out_hbm.at[pl.ds(base_b * _D, _CHUNK * _D)], sem_b)
    return carry

  lax.fori_loop(0, _NPAIRS, body, 0)
  pltpu.make_async_copy(o_a, out_hbm.at[pl.ds(0, _CHUNK * _D)], sem_a).wait()
  pltpu.make_async_copy(o_b, out_hbm.at[pl.ds(0, _CHUNK * _D)], sem_b).wait()


def kernel(x, W0, W1, W2, W3, W4, W5, W6, W7, W8):
  x32 = x.astype(jnp.int32).reshape(-1)
  out = _sc_lookup(x32, W0.reshape(-1), W1.reshape(-1), W2.reshape(-1),
                   W3.reshape(-1), W4.reshape(-1), W5.reshape(-1),
                   W6.reshape(-1), W7.reshape(-1), W8.reshape(-1))
  return out.reshape(_N, _D)
